# Initial kernel scaffold; baseline (speedup 1.0000x reference)
import jax
import jax.numpy as jnp
from jax.experimental import pallas as pl


def kernel(res_x, user_x, res_ei0, res_ei1, res_ei2, user_ei0, user_ei1, user_ei2, inverse_idx, res_Wl0, res_bl0, res_Wr0, res_Wl1, res_bl1, res_Wr1, res_Wl2, res_bl2, res_Wr2, user_Wl0, user_bl0, user_Wr0, user_Wl1, user_bl1, user_Wr1, user_Wl2, user_bl2, user_Wr2, Wout, bout):
    raise NotImplementedError("write your pallas kernel here")


# speedup vs baseline: 2.9190x; 2.9190x over previous
import functools

import jax
import jax.numpy as jnp
from jax import lax
from jax.experimental import pallas as pl
from jax.experimental.pallas import tpu as pltpu
from jax.experimental.pallas import tpu_sc as plsc

_D = 128
_NS = 16
_NC = 2


def _rup(x, m):
    return (x + m - 1) // m * m


@functools.cache
def _make_seg_sum(E, ndst):
    assert E % 128 == 0
    G = E // 128
    Hc = _rup((ndst + 1) // 2, 128)
    Ha = Hc + 256
    base_g, rem_g = G // _NS, G % _NS
    zs = Ha // _NS
    rpt = Hc // _NS

    mesh = plsc.VectorSubcoreMesh(core_axis_name="c", subcore_axis_name="s")

    @functools.partial(
        pl.kernel,
        mesh=mesh,
        out_type=[
            jax.ShapeDtypeStruct((2 * Hc, _D), jnp.float32),
            jax.ShapeDtypeStruct((2 * Hc,), jnp.float32),
        ],
        scratch_types=[
            pltpu.VMEM((128,), jnp.int32),
            pltpu.VMEM((128,), jnp.int32),
            pltpu.VMEM((1, 128), jnp.int32),
            pltpu.VMEM((128,), jnp.float32),
            pltpu.VMEM((128, _D), jnp.float32),
            pltpu.VMEM((zs,), jnp.float32),
            pltpu.VMEM_SHARED((Ha, _D), jnp.float32),
            pltpu.VMEM_SHARED((Ha,), jnp.float32),
            pltpu.SemaphoreType.DMA,
        ],
    )
    def seg(x_hbm, src_hbm, dst_hbm, zr_hbm, s_out, c_out,
            src_v, draw_v, didx_v, ones_v, rows_v, cz_v, s_sh, c_sh, sem):
        c = lax.axis_index("c")
        s = lax.axis_index("s")
        pltpu.sync_copy(zr_hbm.at[pl.ds(s * zs, zs)], s_sh.at[pl.ds(s * zs, zs)])

        def zbody(i, carry):
            cz_v[pl.ds(i * 16, 16)] = jnp.zeros((16,), jnp.float32)
            return carry

        lax.fori_loop(0, zs // 16, zbody, 0)
        pltpu.sync_copy(cz_v, c_sh.at[pl.ds(s * zs, zs)])
        for t in range(8):
            ones_v[pl.ds(16 * t, 16)] = jnp.full((16,), 1.0, jnp.float32)
        plsc.subcore_barrier()

        start = s * base_g + jnp.minimum(s, rem_g)
        n = base_g + jnp.where(s < rem_g, 1, 0)
        lo_base = c * Hc

        def body(i, carry):
            off = (start + i) * 128
            pltpu.sync_copy(src_hbm.at[pl.ds(off, 128)], src_v)
            pltpu.sync_copy(dst_hbm.at[pl.ds(off, 128)], draw_v)
            for t in range(8):
                d = draw_v[pl.ds(16 * t, 16)]
                lo = d - lo_base
                m = (lo >= 0) & (lo < Hc)
                didx_v[0, pl.ds(16 * t, 16)] = jnp.where(m, lo, Hc + (d & 63))
            pltpu.async_copy(x_hbm.at[src_v], rows_v, sem).wait()
            pltpu.sync_copy(rows_v, s_sh.at[didx_v.at[0]], add=True)
            pltpu.sync_copy(ones_v, c_sh.at[didx_v.at[0]], add=True)
            return carry

        lax.fori_loop(0, n, body, 0)
        plsc.subcore_barrier()
        pltpu.sync_copy(s_sh.at[pl.ds(s * rpt, rpt)],
                        s_out.at[pl.ds(c * Hc + s * rpt, rpt)])
        pltpu.sync_copy(c_sh.at[pl.ds(s * rpt, rpt)], cz_v.at[pl.ds(0, rpt)])
        pltpu.sync_copy(cz_v.at[pl.ds(0, rpt)],
                        c_out.at[pl.ds(c * Hc + s * rpt, rpt)])

    return seg, Hc, Ha


@functools.cache
def _make_pool():
    mesh = plsc.VectorSubcoreMesh(core_axis_name="c", subcore_axis_name="s")

    @functools.partial(
        pl.kernel,
        mesh=mesh,
        out_type=jax.ShapeDtypeStruct((2048, _D), jnp.float32),
        scratch_types=[
            pltpu.VMEM((128,), jnp.int32),
            pltpu.VMEM((1, 128), jnp.int32),
            pltpu.VMEM((128, _D), jnp.float32),
            pltpu.VMEM_SHARED((1024, _D), jnp.float32),
            pltpu.SemaphoreType.DMA,
        ],
    )
    def pool(x_hbm, idx_hbm, zp_hbm, u_out, idx_v, oi_v, rows_v, o_sh, sem):
        c = lax.axis_index("c")
        s = lax.axis_index("s")
        pltpu.sync_copy(zp_hbm.at[pl.ds(s * 64, 64)], o_sh.at[pl.ds(s * 64, 64)])
        plsc.subcore_barrier()
        p0 = c * 8192 + s * 512
        lane8 = jnp.arange(16, dtype=jnp.int32) >> 3
        for b in range(4):
            pltpu.sync_copy(idx_hbm.at[pl.ds(p0 + b * 128, 128)], idx_v)
            for t in range(8):
                oi_v[0, pl.ds(16 * t, 16)] = s * 64 + b * 16 + 2 * t + lane8
            pltpu.async_copy(x_hbm.at[idx_v], rows_v, sem).wait()
            pltpu.sync_copy(rows_v, o_sh.at[oi_v.at[0]], add=True)
        plsc.subcore_barrier()
        pltpu.sync_copy(o_sh.at[pl.ds(s * 64, 64)],
                        u_out.at[pl.ds(c * 1024 + s * 64, 64)])

    return pool


def _dense(S, C2, X, Wl, bl2, Wr, ndst, relu):
    B = 512 if ndst >= 512 else ndst

    def body(s_ref, c_ref, x_ref, wl_ref, bl_ref, wr_ref, o_ref):
        cnt = jnp.maximum(c_ref[...], 1.0)
        mean = s_ref[...] / cnt
        acc = lax.dot_general(mean, wl_ref[...], (((1,), (1,)), ((), ())),
                              preferred_element_type=jnp.float32)
        acc = acc + lax.dot_general(x_ref[...], wr_ref[...],
                                    (((1,), (1,)), ((), ())),
                                    preferred_element_type=jnp.float32)
        acc = acc + bl_ref[...]
        if relu:
            acc = jnp.maximum(acc, 0.0)
        o_ref[...] = acc

    return pl.pallas_call(
        body,
        grid=(pl.cdiv(ndst, B),),
        in_specs=[
            pl.BlockSpec((B, _D), lambda i: (i, 0)),
            pl.BlockSpec((B, 1), lambda i: (i, 0)),
            pl.BlockSpec((B, _D), lambda i: (i, 0)),
            pl.BlockSpec((_D, _D), lambda i: (0, 0)),
            pl.BlockSpec((1, _D), lambda i: (0, 0)),
            pl.BlockSpec((_D, _D), lambda i: (0, 0)),
        ],
        out_specs=pl.BlockSpec((B, _D), lambda i: (i, 0)),
        out_shape=jax.ShapeDtypeStruct((ndst, _D), jnp.float32),
    )(S, C2, X, Wl, bl2, Wr)


def _final(res3, usum, wa, wb, bout2):
    def body(r_ref, u_ref, wa_ref, wb_ref, b_ref, o_ref):
        o = lax.dot_general(r_ref[...], wa_ref[...], (((1,), (1,)), ((), ())),
                            preferred_element_type=jnp.float32)
        o = o + lax.dot_general(u_ref[...] * 0.125, wb_ref[...],
                                (((1,), (1,)), ((), ())),
                                preferred_element_type=jnp.float32)
        o_ref[...] = o + b_ref[...]

    return pl.pallas_call(
        body,
        out_shape=jax.ShapeDtypeStruct((2048, 64), jnp.float32),
    )(res3, usum, wa, wb, bout2)


def _gnn_side(x, eis, sizes, params):
    h = x
    for l in range(3):
        ei = eis[l]
        ndst = sizes[l]
        wl, bl, wr = params[l]
        seg, hc, ha = _make_seg_sum(ei.shape[1], ndst)
        zr = jnp.zeros((ha, _D), jnp.float32)
        S, C = seg(h, ei[0], ei[1], zr)
        h = _dense(S, C.reshape(-1, 1), h, wl, bl.reshape(1, -1), wr,
                   ndst, relu=(l != 1))
    return h


def kernel(res_x, user_x, res_ei0, res_ei1, res_ei2, user_ei0, user_ei1,
           user_ei2, inverse_idx, res_Wl0, res_bl0, res_Wr0, res_Wl1,
           res_bl1, res_Wr1, res_Wl2, res_bl2, res_Wr2, user_Wl0, user_bl0,
           user_Wr0, user_Wl1, user_bl1, user_Wr1, user_Wl2, user_bl2,
           user_Wr2, Wout, bout):
    rp = ((res_Wl0, res_bl0, res_Wr0), (res_Wl1, res_bl1, res_Wr1),
          (res_Wl2, res_bl2, res_Wr2))
    up = ((user_Wl0, user_bl0, user_Wr0), (user_Wl1, user_bl1, user_Wr1),
          (user_Wl2, user_bl2, user_Wr2))
    res3 = _gnn_side(res_x, (res_ei0, res_ei1, res_ei2), (25000, 6000, 2048), rp)
    usr3 = _gnn_side(user_x, (user_ei0, user_ei1, user_ei2), (25000, 6000, 4096), up)
    pool = _make_pool()
    zp = jnp.zeros((1024, _D), jnp.float32)
    u_sum = pool(usr3, inverse_idx, zp)
    return _final(res3, u_sum, Wout[:, :_D], Wout[:, _D:],
                  bout.reshape(1, -1))
